# baseline (speedup 1.0000x reference)
import functools

import jax
import jax.numpy as jnp
from jax.experimental import pallas as pl
from jax.experimental.pallas import tpu as pltpu

HIDDEN = 4096
NUM_EXPERTS = 64
TOP_K = 8
BLOCK_M = 512
NEG = -3.0e38


def _router_body(wt_ref, x_ref, w_out, i_out, counts_ref):
    logits = jax.lax.dot_general(
        wt_ref[...], x_ref[...], (((1,), (1,)), ((), ())),
        preferred_element_type=jnp.float32,
    )

    iota_e = jax.lax.broadcasted_iota(jnp.int32, logits.shape, 0)
    work = logits
    vals = []
    idxs = []
    mask_acc = jnp.zeros_like(logits)
    for _ in range(TOP_K):
        m = jnp.max(work, axis=0, keepdims=True)
        is_max = work == m
        idx = jnp.min(jnp.where(is_max, iota_e, NUM_EXPERTS), axis=0, keepdims=True)
        chosen = iota_e == idx
        mask_acc = mask_acc + chosen.astype(jnp.float32)
        work = jnp.where(chosen, NEG, work)
        vals.append(m)
        idxs.append(idx)

    v = jnp.concatenate(vals, axis=0)
    e = jnp.exp(v - v[0:1, :])
    w_out[...] = e / jnp.sum(e, axis=0, keepdims=True)
    i_out[...] = jnp.concatenate(idxs, axis=0)

    counts_ref[...] = jnp.sum(mask_acc, axis=1, keepdims=True).reshape(1, 1, NUM_EXPERTS)


def _aux_body(counts_ref, aux_ref, *, n_tokens):
    c = jnp.sum(counts_ref[...], axis=0, keepdims=True)
    meanv = c / n_tokens
    mu = jnp.sum(meanv, keepdims=True) / NUM_EXPERTS
    d = meanv - mu
    aux_ref[...] = jnp.sum(d * d, keepdims=True) * NUM_EXPERTS / (NUM_EXPERTS - 1)


def kernel(x, W):
    b, s, h = x.shape
    n_tokens = b * s
    x_flat = x.reshape(n_tokens, h)
    n_steps = n_tokens // BLOCK_M

    w_t, i_t, counts = pl.pallas_call(
        _router_body,
        grid=(n_steps,),
        in_specs=[
            pl.BlockSpec((NUM_EXPERTS, h), lambda i: (0, 0)),
            pl.BlockSpec((BLOCK_M, h), lambda i: (i, 0)),
        ],
        out_specs=[
            pl.BlockSpec((TOP_K, BLOCK_M), lambda i: (0, i)),
            pl.BlockSpec((TOP_K, BLOCK_M), lambda i: (0, i)),
            pl.BlockSpec((1, 1, NUM_EXPERTS), lambda i: (i, 0, 0)),
        ],
        out_shape=[
            jax.ShapeDtypeStruct((TOP_K, n_tokens), jnp.float32),
            jax.ShapeDtypeStruct((TOP_K, n_tokens), jnp.int32),
            jax.ShapeDtypeStruct((n_steps, 1, NUM_EXPERTS), jnp.float32),
        ],
        compiler_params=pltpu.CompilerParams(
            dimension_semantics=("arbitrary",),
        ),
    )(W, x_flat)

    aux = pl.pallas_call(
        functools.partial(_aux_body, n_tokens=float(n_tokens)),
        out_shape=jax.ShapeDtypeStruct((1, 1), jnp.float32),
    )(counts.reshape(n_steps, NUM_EXPERTS))

    return w_t.T, i_t.T, aux[0, 0]


# speedup vs baseline: 1.0974x; 1.0974x over previous
import functools

import jax
import jax.numpy as jnp
from jax.experimental import pallas as pl

HIDDEN = 4096
NUM_EXPERTS = 64
TOP_K = 8
BLOCK_M = 1024
NEG = -3.0e38


def _router_body(wt_ref, x_ref, w_out, i_out, counts_ref, aux_ref, *, n_tokens, n_steps):
    step = pl.program_id(0)

    @pl.when(step == 0)
    def _init():
        counts_ref[...] = jnp.zeros_like(counts_ref)

    logits = jax.lax.dot_general(
        wt_ref[...], x_ref[...], (((1,), (1,)), ((), ())),
        preferred_element_type=jnp.float32,
    )

    iota_e = jax.lax.broadcasted_iota(jnp.int32, logits.shape, 0)
    work = logits
    vals = []
    idxs = []
    mask_acc = jnp.zeros_like(logits)
    for _ in range(TOP_K):
        m = jnp.max(work, axis=0, keepdims=True)
        is_max = work == m
        idx = jnp.min(jnp.where(is_max, iota_e, NUM_EXPERTS), axis=0, keepdims=True)
        chosen = iota_e == idx
        mask_acc = mask_acc + chosen.astype(jnp.float32)
        work = jnp.where(chosen, NEG, work)
        vals.append(m)
        idxs.append(idx)

    v = jnp.concatenate(vals, axis=0)
    e = jnp.exp(v - v[0:1, :])
    w_out[...] = e / jnp.sum(e, axis=0, keepdims=True)
    i_out[...] = jnp.concatenate(idxs, axis=0)

    counts_ref[...] += jnp.sum(mask_acc, axis=1, keepdims=True).reshape(1, NUM_EXPERTS)

    @pl.when(step == n_steps - 1)
    def _finish():
        meanv = counts_ref[...] / n_tokens
        mu = jnp.sum(meanv, keepdims=True) / NUM_EXPERTS
        d = meanv - mu
        aux_ref[...] = jnp.sum(d * d, keepdims=True) * NUM_EXPERTS / (NUM_EXPERTS - 1)


def kernel(x, W):
    b, s, h = x.shape
    n_tokens = b * s
    x_flat = x.reshape(n_tokens, h)
    n_steps = n_tokens // BLOCK_M

    body = functools.partial(_router_body, n_tokens=float(n_tokens), n_steps=n_steps)
    w_t, i_t, _counts, aux = pl.pallas_call(
        body,
        grid=(n_steps,),
        in_specs=[
            pl.BlockSpec((NUM_EXPERTS, h), lambda i: (0, 0)),
            pl.BlockSpec((BLOCK_M, h), lambda i: (i, 0)),
        ],
        out_specs=[
            pl.BlockSpec((TOP_K, BLOCK_M), lambda i: (0, i)),
            pl.BlockSpec((TOP_K, BLOCK_M), lambda i: (0, i)),
            pl.BlockSpec((1, NUM_EXPERTS), lambda i: (0, 0)),
            pl.BlockSpec((1, 1), lambda i: (0, 0)),
        ],
        out_shape=[
            jax.ShapeDtypeStruct((TOP_K, n_tokens), jnp.float32),
            jax.ShapeDtypeStruct((TOP_K, n_tokens), jnp.int32),
            jax.ShapeDtypeStruct((1, NUM_EXPERTS), jnp.float32),
            jax.ShapeDtypeStruct((1, 1), jnp.float32),
        ],
    )(W, x_flat)

    return w_t.T, i_t.T, aux[0, 0]
